# baseline (speedup 1.0000x reference)
import jax
import jax.numpy as jnp
from jax import lax
from jax.experimental import pallas as pl
from jax.experimental.pallas import tpu as pltpu
from jax.experimental.pallas import tpu_sc as plsc

B = 16384
K = 64
NC = 2
NS = 16
NW = NC * NS
RPW = B // NW
CHUNK = 128
NCHUNK = RPW // CHUNK
GROUP = 16


def _mf_body(uid_hbm, iid_hbm, user_hbm, item_hbm, bu_hbm, bi_hbm, bias_hbm,
             out_hbm,
             uid_v, iid_v, u_rows, v_rows, bu_v, bi_v, out_v, bias_v, tile_v,
             sem_idx, sem_rows, sem_bias):
  c = lax.axis_index("c")
  s = lax.axis_index("s")
  wid = s * NC + c
  base = wid * RPW

  idx_copies = []
  for j in range(NCHUNK):
    idx_copies.append(pltpu.make_async_copy(
        uid_hbm.at[pl.ds(base + j * CHUNK, CHUNK)], uid_v.at[j], sem_idx))
    idx_copies.append(pltpu.make_async_copy(
        iid_hbm.at[pl.ds(base + j * CHUNK, CHUNK)], iid_v.at[j], sem_idx))
  idx_copies.append(pltpu.make_async_copy(bias_hbm, bias_v, sem_idx))
  for cp in idx_copies:
    cp.start()
  for cp in idx_copies:
    cp.wait()

  row_copies = []
  for j in range(NCHUNK):
    sl = pl.ds(j * CHUNK, CHUNK)
    row_copies.append(pltpu.make_async_copy(
        user_hbm.at[uid_v.at[j]], u_rows.at[sl], sem_rows))
    row_copies.append(pltpu.make_async_copy(
        item_hbm.at[iid_v.at[j]], v_rows.at[sl], sem_rows))
    row_copies.append(pltpu.make_async_copy(
        bu_hbm.at[uid_v.at[j]], bu_v.at[sl], sem_bias))
    row_copies.append(pltpu.make_async_copy(
        bi_hbm.at[iid_v.at[j]], bi_v.at[sl], sem_bias))
  for cp in row_copies:
    cp.start()
  for cp in row_copies:
    cp.wait()

  bias_vec = bias_v[...]
  lane = lax.iota(jnp.int32, 16)

  def group_body(g, carry):
    rbase = g * GROUP
    for r in range(GROUP):
      row = rbase + r
      acc = u_rows[row, pl.ds(0, 16)] * v_rows[row, pl.ds(0, 16)]
      for cb in range(1, K // 16):
        acc = acc + (u_rows[row, pl.ds(cb * 16, 16)] *
                     v_rows[row, pl.ds(cb * 16, 16)])
      tile_v[pl.ds(r * 16, 16)] = acc
    dotv = plsc.load_gather(tile_v, [lane * 16])
    for cb in range(1, 16):
      dotv = dotv + plsc.load_gather(tile_v, [lane * 16 + cb])
    sl = pl.ds(rbase, GROUP)
    out_v[sl] = dotv + bu_v[sl] + bi_v[sl] + bias_vec
    return carry

  lax.fori_loop(0, RPW // GROUP, group_body, 0)

  pltpu.sync_copy(out_v, out_hbm.at[pl.ds(base, RPW)])


@jax.jit
def kernel(train_x, user_w, item_w, bias_user_w, bias_item_w, bias):
  uid = train_x[:, 0]
  iid = train_x[:, 1]
  bu = bias_user_w.reshape(-1)
  bi = bias_item_w.reshape(-1)
  bias16 = jnp.broadcast_to(bias, (16,))
  mesh = plsc.VectorSubcoreMesh(core_axis_name="c", subcore_axis_name="s",
                                num_cores=NC, num_subcores=NS)
  fn = pl.kernel(
      _mf_body,
      out_type=jax.ShapeDtypeStruct((B,), jnp.float32),
      mesh=mesh,
      compiler_params=pltpu.CompilerParams(needs_layout_passes=False,
                                           use_tc_tiling_on_sc=False),
      scratch_types=[
          pltpu.VMEM((NCHUNK, CHUNK), jnp.int32),
          pltpu.VMEM((NCHUNK, CHUNK), jnp.int32),
          pltpu.VMEM((RPW, K), jnp.float32),
          pltpu.VMEM((RPW, K), jnp.float32),
          pltpu.VMEM((RPW,), jnp.float32),
          pltpu.VMEM((RPW,), jnp.float32),
          pltpu.VMEM((RPW,), jnp.float32),
          pltpu.VMEM((16,), jnp.float32),
          pltpu.VMEM((GROUP * 16,), jnp.float32),
          pltpu.SemaphoreType.DMA,
          pltpu.SemaphoreType.DMA,
          pltpu.SemaphoreType.DMA,
      ],
  )
  return fn(uid, iid, user_w, item_w, bu, bi, bias16)
